# baseline (speedup 1.0000x reference)
import jax
import jax.numpy as jnp
from jax import lax
from jax.experimental import pallas as pl
from jax.experimental.pallas import tpu as pltpu
from jax.experimental.pallas import tpu_sc as plsc

N = 100000
PRUNE_MAX_DEGREE = 100
NC = 2
NS = 16
NW = NC * NS
L = 16
ROW = 128
CH = 4000
CHA = 10000
RING = 4

NPAD = ((N + NW * 8 - 1) // (NW * 8)) * (NW * 8)
PS = NPAD // NS


def _worker_id():
    return lax.axis_index("s") * NC + lax.axis_index("c")


def _zero_fill(buf, n, dtype):
    def zb(i, carry):
        buf[pl.ds(i * L, L)] = jnp.zeros((L,), dtype)
        return carry
    lax.fori_loop(0, n // L, zb, 0)


class _Board:
    def __init__(self):
        self._p = {}

    def put(self, key, descs):
        self._p[key] = descs

    def wait(self, key):
        for d in self._p.pop(key, ()):
            d.wait()

    def drain(self):
        for key in list(self._p):
            self.wait(key)


def _deg_body(eif, out, *rest):
    src_v = rest[0:RING]
    dst_v = rest[RING:2 * RING]
    ones_v, tmp_v = rest[2 * RING:2 * RING + 2]
    sem_ld = rest[2 * RING + 2:2 * RING + 2 + RING]
    sem_sc = rest[2 * RING + 2 + RING:2 * RING + 2 + 2 * RING]
    deg_sh = rest[-1]
    c = lax.axis_index("c")
    s = lax.axis_index("s")
    w = _worker_id()
    E = eif.shape[0] // 2
    nck = E // CHA // NW

    def ob(i, carry):
        ones_v[pl.ds(i * L, L)] = jnp.ones((L,), jnp.int32)
        return carry
    lax.fori_loop(0, CHA // L, ob, 0)
    _zero_fill(tmp_v, PS, jnp.int32)
    pltpu.sync_copy(tmp_v, deg_sh.at[pl.ds(s * PS, PS)])
    plsc.subcore_barrier()

    lo = w * nck
    ld, sc = _Board(), _Board()

    def start_loads(g):
        b = g % RING
        e0 = (lo + g) * CHA
        ld.put(g, [
            pltpu.async_copy(eif.at[pl.ds(e0, CHA)], src_v[b], sem_ld[b]),
            pltpu.async_copy(eif.at[pl.ds(E + e0, CHA)], dst_v[b],
                             sem_ld[b]),
        ])

    for r in range(min(RING - 1, nck)):
        start_loads(r)
    for g in range(nck):
        b = g % RING
        ld.wait(g)
        sc.put(g, [
            pltpu.async_copy(ones_v, deg_sh.at[src_v[b]], sem_sc[b],
                             add=True),
            pltpu.async_copy(ones_v, deg_sh.at[dst_v[b]], sem_sc[b],
                             add=True),
        ])
        if g + RING - 1 < nck:
            sc.wait(g - 1)
            start_loads(g + RING - 1)
    sc.drain()
    plsc.subcore_barrier()
    pltpu.sync_copy(deg_sh.at[pl.ds(s * PS, PS)], tmp_v)
    pltpu.sync_copy(tmp_v, out.at[pl.ds(c * NPAD + s * PS, PS)])


def _main_body(eif, et1, lg1, relp, ht, degp, out, kn_out, *rest):
    src_v = rest[0:RING]
    dst_v = rest[RING:2 * RING]
    et_v = rest[2 * RING:3 * RING]
    lg_v = rest[3 * RING:4 * RING]
    ks_v = rest[4 * RING:4 * RING + 2]
    msg_v = rest[4 * RING + 2:4 * RING + 4]
    rel_v, ht_v, d0_v, d1_v, kn_v = rest[4 * RING + 4:4 * RING + 9]
    p = 4 * RING + 9
    sem_ld = rest[p:p + RING]
    sem_gt = rest[p + RING:p + RING + 2]
    sem_sc = rest[p + RING + 2:p + RING + 4]
    kn_sh, acc_sh = rest[p + RING + 4:]
    c = lax.axis_index("c")
    s = lax.axis_index("s")
    w = _worker_id()
    E = et1.shape[0]
    nck = E // CH // NW

    pltpu.sync_copy(relp, rel_v)
    pltpu.sync_copy(ht, ht_v)

    off = s * PS
    pltpu.sync_copy(degp.at[pl.ds(off, PS)], d0_v)
    pltpu.sync_copy(degp.at[pl.ds(NPAD + off, PS)], d1_v)
    head = ht_v[0, :]
    tail = ht_v[1, :]

    def b1(i, carry):
        sl = pl.ds(i * L, L)
        dd = d0_v[sl] + d1_v[sl]
        nid = off + i * L + lax.iota(jnp.int32, L)
        keep = ((dd <= PRUNE_MAX_DEGREE) | (nid == head)) | (nid == tail)
        kn_v[sl] = jnp.where(keep, 1.0, 0.0).astype(jnp.float32)
        return carry

    lax.fori_loop(0, PS // L, b1, 0)
    pltpu.sync_copy(kn_v, kn_sh.at[pl.ds(off, PS)])

    @pl.when(c == 0)
    def _():
        pltpu.sync_copy(kn_v, kn_out.at[pl.ds(off, PS)])

    _zero_fill(kn_v, PS, jnp.float32)
    pltpu.sync_copy(kn_v, acc_sh.at[pl.ds(off, PS)])
    plsc.subcore_barrier()

    lo = w * nck
    ld, gt, sc = _Board(), _Board(), _Board()

    def start_loads(g):
        b = g % RING
        e0 = (lo + g) * CH
        ld.put(g, [
            pltpu.async_copy(eif.at[pl.ds(e0, CH)], src_v[b], sem_ld[b]),
            pltpu.async_copy(eif.at[pl.ds(E + e0, CH)], dst_v[b],
                             sem_ld[b]),
            pltpu.async_copy(et1.at[pl.ds(e0, CH)], et_v[b], sem_ld[b]),
            pltpu.async_copy(lg1.at[pl.ds(e0, CH)], lg_v[b], sem_ld[b]),
        ])

    def start_gathers(g):
        bi, bg = g % RING, g % 2
        gt.put(g, [
            pltpu.async_copy(kn_sh.at[src_v[bi]], ks_v[bg], sem_gt[bg]),
        ])

    for r in range(min(RING - 1, nck)):
        start_loads(r)
    if nck:
        ld.wait(0)
        start_gathers(0)
    for g in range(nck):
        bi, bg = g % RING, g % 2
        gt.wait(g)
        if g + 1 < nck:
            ld.wait(g + 1)
            start_gathers(g + 1)
        sc.wait(g - 2)

        def cb(j, carry2):
            for u in range(2):
                sl = pl.ds(j * 2 * L + u * L, L)
                lg = lg_v[bi][sl]
                sig = 1.0 / (1.0 + jnp.exp(-lg))
                re = plsc.load_gather(rel_v, [et_v[bi][sl]])
                msg_v[bg][sl] = sig * ks_v[bg][sl] * re
            return carry2

        lax.fori_loop(0, CH // (2 * L), cb, 0)
        sc.put(g, [pltpu.async_copy(msg_v[bg], acc_sh.at[dst_v[bi]],
                                    sem_sc[bg], add=True)])
        if g + RING - 1 < nck:
            sc.wait(g - 1)
            start_loads(g + RING - 1)
    sc.drain()
    plsc.subcore_barrier()
    pltpu.sync_copy(acc_sh.at[pl.ds(off, PS)], kn_v)
    pltpu.sync_copy(kn_v, out.at[pl.ds(c * NPAD + off, PS)])


def _combine_body(p_ref, k_ref, o_ref):
    o_ref[...] = (p_ref[0] + p_ref[1]) * k_ref[...]


def kernel(edge_index, edge_type, edge_mask_logits, rel_emb, head_idx, tail_idx):
    E = edge_type.shape[0]
    R = rel_emb.shape[0]
    assert E % (CH * NW) == 0
    eif = edge_index.reshape(2 * E)

    rpad = ((R + L - 1) // L) * L
    relp = jnp.zeros((rpad,), jnp.float32).at[:R].set(rel_emb)
    ht = jnp.stack([jnp.full((L,), head_idx, jnp.int32),
                    jnp.full((L,), tail_idx, jnp.int32)])

    mesh = plsc.VectorSubcoreMesh(core_axis_name="c", subcore_axis_name="s")

    deg_partial = pl.kernel(
        _deg_body,
        out_type=jax.ShapeDtypeStruct((NC * NPAD,), jnp.int32),
        mesh=mesh,
        scratch_types=(
            [pltpu.VMEM((CHA,), jnp.int32)] * RING
            + [pltpu.VMEM((CHA,), jnp.int32)] * RING
            + [
                pltpu.VMEM((CHA,), jnp.int32),
                pltpu.VMEM((PS,), jnp.int32),
            ]
            + [pltpu.SemaphoreType.DMA] * RING
            + [pltpu.SemaphoreType.DMA] * RING
            + [pltpu.VMEM_SHARED((NPAD,), jnp.int32)]
        ),
    )(eif)

    out_partial, kn_arr = pl.kernel(
        _main_body,
        out_type=(jax.ShapeDtypeStruct((NC * NPAD,), jnp.float32),
                  jax.ShapeDtypeStruct((NPAD,), jnp.float32)),
        mesh=mesh,
        scratch_types=(
            [pltpu.VMEM((CH,), jnp.int32)] * RING
            + [pltpu.VMEM((CH,), jnp.int32)] * RING
            + [pltpu.VMEM((CH,), jnp.int32)] * RING
            + [pltpu.VMEM((CH,), jnp.float32)] * RING
            + [pltpu.VMEM((CH,), jnp.float32)] * 2
            + [pltpu.VMEM((CH,), jnp.float32)] * 2
            + [
                pltpu.VMEM((rpad,), jnp.float32),
                pltpu.VMEM((2, L), jnp.int32),
                pltpu.VMEM((PS,), jnp.int32),
                pltpu.VMEM((PS,), jnp.int32),
                pltpu.VMEM((PS,), jnp.float32),
            ]
            + [pltpu.SemaphoreType.DMA] * RING
            + [pltpu.SemaphoreType.DMA] * 2
            + [pltpu.SemaphoreType.DMA] * 2
            + [pltpu.VMEM_SHARED((NPAD,), jnp.float32),
               pltpu.VMEM_SHARED((NPAD,), jnp.float32)]
        ),
        compiler_params=pltpu.CompilerParams(needs_layout_passes=False),
    )(eif, edge_type, edge_mask_logits, relp, ht, deg_partial)

    out = pl.pallas_call(
        _combine_body,
        out_shape=jax.ShapeDtypeStruct((NPAD // ROW, ROW), jnp.float32),
    )(out_partial.reshape(NC, NPAD // ROW, ROW),
      kn_arr.reshape(NPAD // ROW, ROW))
    return out.reshape(-1)[:N]


# speedup vs baseline: 1.0051x; 1.0051x over previous
import jax
import jax.numpy as jnp
from jax import lax
from jax.experimental import pallas as pl
from jax.experimental.pallas import tpu as pltpu
from jax.experimental.pallas import tpu_sc as plsc

N = 100000
PRUNE_MAX_DEGREE = 100
NC = 2
NS = 16
NW = NC * NS
L = 16
ROW = 128
CH = 4000
CHA = 10000
RING = 4

NPAD = ((N + NW * 8 - 1) // (NW * 8)) * (NW * 8)
PS = NPAD // NS


def _worker_id():
    return lax.axis_index("s") * NC + lax.axis_index("c")


def _zero_fill(buf, n, dtype):
    def zb(i, carry):
        buf[pl.ds(i * L, L)] = jnp.zeros((L,), dtype)
        return carry
    lax.fori_loop(0, n // L, zb, 0)


class _Board:
    def __init__(self):
        self._p = {}

    def put(self, key, descs):
        self._p[key] = descs

    def wait(self, key):
        for d in self._p.pop(key, ()):
            d.wait()

    def drain(self):
        for key in list(self._p):
            self.wait(key)


def _deg_body(eif, out, *rest):
    idx_v = rest[0:RING]
    ones_v, tmp_v = rest[RING:RING + 2]
    sem_ld = rest[RING + 2:RING + 2 + RING]
    sem_sc = rest[RING + 2 + RING:RING + 2 + 2 * RING]
    deg_sh = rest[-1]
    c = lax.axis_index("c")
    s = lax.axis_index("s")
    w = _worker_id()
    nck = eif.shape[0] // CHA // NW

    def ob(i, carry):
        ones_v[pl.ds(i * L, L)] = jnp.ones((L,), jnp.int32)
        return carry
    lax.fori_loop(0, CHA // L, ob, 0)
    _zero_fill(tmp_v, PS, jnp.int32)
    pltpu.sync_copy(tmp_v, deg_sh.at[pl.ds(s * PS, PS)])
    plsc.subcore_barrier()

    lo = w * nck
    ld, sc = _Board(), _Board()

    def start_loads(g):
        b = g % RING
        e0 = (lo + g) * CHA
        ld.put(g, [
            pltpu.async_copy(eif.at[pl.ds(e0, CHA)], idx_v[b], sem_ld[b]),
        ])

    for r in range(min(RING - 1, nck)):
        start_loads(r)
    for g in range(nck):
        b = g % RING
        ld.wait(g)
        sc.put(g, [
            pltpu.async_copy(ones_v, deg_sh.at[idx_v[b]], sem_sc[b],
                             add=True),
        ])
        if g + RING - 1 < nck:
            sc.wait(g - 1)
            start_loads(g + RING - 1)
    sc.drain()
    plsc.subcore_barrier()
    pltpu.sync_copy(deg_sh.at[pl.ds(s * PS, PS)], tmp_v)
    pltpu.sync_copy(tmp_v, out.at[pl.ds(c * NPAD + s * PS, PS)])


def _main_body(eif, et1, lg1, relp, ht, degp, out, kn_out, *rest):
    src_v = rest[0:RING]
    dst_v = rest[RING:2 * RING]
    et_v = rest[2 * RING:3 * RING]
    lg_v = rest[3 * RING:4 * RING]
    ks_v = rest[4 * RING:4 * RING + 2]
    msg_v = rest[4 * RING + 2:4 * RING + 4]
    rel_v, ht_v, d0_v, d1_v, kn_v = rest[4 * RING + 4:4 * RING + 9]
    p = 4 * RING + 9
    sem_ld = rest[p:p + RING]
    sem_gt = rest[p + RING:p + RING + 2]
    sem_sc = rest[p + RING + 2:p + RING + 4]
    kn_sh, acc_sh = rest[p + RING + 4:]
    c = lax.axis_index("c")
    s = lax.axis_index("s")
    w = _worker_id()
    E = et1.shape[0]
    nck = E // CH // NW

    pltpu.sync_copy(relp, rel_v)
    pltpu.sync_copy(ht, ht_v)

    off = s * PS
    pltpu.sync_copy(degp.at[pl.ds(off, PS)], d0_v)
    pltpu.sync_copy(degp.at[pl.ds(NPAD + off, PS)], d1_v)
    head = ht_v[0, :]
    tail = ht_v[1, :]

    def b1(i, carry):
        sl = pl.ds(i * L, L)
        dd = d0_v[sl] + d1_v[sl]
        nid = off + i * L + lax.iota(jnp.int32, L)
        keep = ((dd <= PRUNE_MAX_DEGREE) | (nid == head)) | (nid == tail)
        kn_v[sl] = jnp.where(keep, 1.0, 0.0).astype(jnp.float32)
        return carry

    lax.fori_loop(0, PS // L, b1, 0)
    pltpu.sync_copy(kn_v, kn_sh.at[pl.ds(off, PS)])

    @pl.when(c == 0)
    def _():
        pltpu.sync_copy(kn_v, kn_out.at[pl.ds(off, PS)])

    _zero_fill(kn_v, PS, jnp.float32)
    pltpu.sync_copy(kn_v, acc_sh.at[pl.ds(off, PS)])
    plsc.subcore_barrier()

    lo = w * nck
    ld, gt, sc = _Board(), _Board(), _Board()

    def start_loads(g):
        b = g % RING
        e0 = (lo + g) * CH
        ld.put(g, [
            pltpu.async_copy(eif.at[pl.ds(e0, CH)], src_v[b], sem_ld[b]),
            pltpu.async_copy(eif.at[pl.ds(E + e0, CH)], dst_v[b],
                             sem_ld[b]),
            pltpu.async_copy(et1.at[pl.ds(e0, CH)], et_v[b], sem_ld[b]),
            pltpu.async_copy(lg1.at[pl.ds(e0, CH)], lg_v[b], sem_ld[b]),
        ])

    def start_gathers(g):
        bi, bg = g % RING, g % 2
        gt.put(g, [
            pltpu.async_copy(kn_sh.at[src_v[bi]], ks_v[bg], sem_gt[bg]),
        ])

    for r in range(min(RING - 1, nck)):
        start_loads(r)
    if nck:
        ld.wait(0)
        start_gathers(0)
    for g in range(nck):
        bi, bg = g % RING, g % 2
        gt.wait(g)
        if g + 1 < nck:
            ld.wait(g + 1)
            start_gathers(g + 1)
        sc.wait(g - 2)

        def cb(j, carry2):
            sl = pl.ds(j * L, L)
            lg = lg_v[bi][sl]
            sig = 1.0 / (1.0 + jnp.exp(-lg))
            re = plsc.load_gather(rel_v, [et_v[bi][sl]])
            msg_v[bg][sl] = sig * ks_v[bg][sl] * re
            return carry2

        lax.fori_loop(0, CH // L, cb, 0)
        sc.put(g, [pltpu.async_copy(msg_v[bg], acc_sh.at[dst_v[bi]],
                                    sem_sc[bg], add=True)])
        if g + RING - 1 < nck:
            sc.wait(g - 1)
            start_loads(g + RING - 1)
    sc.drain()
    plsc.subcore_barrier()
    pltpu.sync_copy(acc_sh.at[pl.ds(off, PS)], kn_v)
    pltpu.sync_copy(kn_v, out.at[pl.ds(c * NPAD + off, PS)])


def _combine_body(p_ref, k_ref, o_ref):
    o_ref[...] = (p_ref[0] + p_ref[1]) * k_ref[...]


def kernel(edge_index, edge_type, edge_mask_logits, rel_emb, head_idx, tail_idx):
    E = edge_type.shape[0]
    R = rel_emb.shape[0]
    assert E % (CH * NW) == 0
    eif = edge_index.reshape(2 * E)

    rpad = ((R + L - 1) // L) * L
    relp = jnp.zeros((rpad,), jnp.float32).at[:R].set(rel_emb)
    ht = jnp.stack([jnp.full((L,), head_idx, jnp.int32),
                    jnp.full((L,), tail_idx, jnp.int32)])

    mesh = plsc.VectorSubcoreMesh(core_axis_name="c", subcore_axis_name="s")

    deg_partial = pl.kernel(
        _deg_body,
        out_type=jax.ShapeDtypeStruct((NC * NPAD,), jnp.int32),
        mesh=mesh,
        scratch_types=(
            [pltpu.VMEM((CHA,), jnp.int32)] * RING
            + [
                pltpu.VMEM((CHA,), jnp.int32),
                pltpu.VMEM((PS,), jnp.int32),
            ]
            + [pltpu.SemaphoreType.DMA] * RING
            + [pltpu.SemaphoreType.DMA] * RING
            + [pltpu.VMEM_SHARED((NPAD,), jnp.int32)]
        ),
    )(eif)

    out_partial, kn_arr = pl.kernel(
        _main_body,
        out_type=(jax.ShapeDtypeStruct((NC * NPAD,), jnp.float32),
                  jax.ShapeDtypeStruct((NPAD,), jnp.float32)),
        mesh=mesh,
        scratch_types=(
            [pltpu.VMEM((CH,), jnp.int32)] * RING
            + [pltpu.VMEM((CH,), jnp.int32)] * RING
            + [pltpu.VMEM((CH,), jnp.int32)] * RING
            + [pltpu.VMEM((CH,), jnp.float32)] * RING
            + [pltpu.VMEM((CH,), jnp.float32)] * 2
            + [pltpu.VMEM((CH,), jnp.float32)] * 2
            + [
                pltpu.VMEM((rpad,), jnp.float32),
                pltpu.VMEM((2, L), jnp.int32),
                pltpu.VMEM((PS,), jnp.int32),
                pltpu.VMEM((PS,), jnp.int32),
                pltpu.VMEM((PS,), jnp.float32),
            ]
            + [pltpu.SemaphoreType.DMA] * RING
            + [pltpu.SemaphoreType.DMA] * 2
            + [pltpu.SemaphoreType.DMA] * 2
            + [pltpu.VMEM_SHARED((NPAD,), jnp.float32),
               pltpu.VMEM_SHARED((NPAD,), jnp.float32)]
        ),
        compiler_params=pltpu.CompilerParams(needs_layout_passes=False),
    )(eif, edge_type, edge_mask_logits, relp, ht, deg_partial)

    out = pl.pallas_call(
        _combine_body,
        out_shape=jax.ShapeDtypeStruct((NPAD // ROW, ROW), jnp.float32),
    )(out_partial.reshape(NC, NPAD // ROW, ROW),
      kn_arr.reshape(NPAD // ROW, ROW))
    return out.reshape(-1)[:N]


# speedup vs baseline: 1.0160x; 1.0109x over previous
import jax
import jax.numpy as jnp
from jax import lax
from jax.experimental import pallas as pl
from jax.experimental.pallas import tpu as pltpu
from jax.experimental.pallas import tpu_sc as plsc

N = 100000
PRUNE_MAX_DEGREE = 100
NC = 2
NS = 16
NW = NC * NS
L = 16
ROW = 128
CH = 4000
RING = 4

NPAD = ((N + NW * 8 - 1) // (NW * 8)) * (NW * 8)
PS = NPAD // NS


def _worker_id():
    return lax.axis_index("s") * NC + lax.axis_index("c")


def _zero_fill(buf, n, dtype):
    def zb(i, carry):
        buf[pl.ds(i * L, L)] = jnp.zeros((L,), dtype)
        return carry
    lax.fori_loop(0, n // L, zb, 0)


class _Board:
    def __init__(self):
        self._p = {}

    def put(self, key, descs):
        self._p[key] = descs

    def wait(self, key):
        for d in self._p.pop(key, ()):
            d.wait()

    def drain(self):
        for key in list(self._p):
            self.wait(key)


def _deg_body(eif, out, *rest):
    src_v = rest[0:RING]
    dst_v = rest[RING:2 * RING]
    ones_v, tmp_v = rest[2 * RING:2 * RING + 2]
    sem_ld = rest[2 * RING + 2:2 * RING + 2 + RING]
    sem_sc = rest[2 * RING + 2 + RING:2 * RING + 2 + 2 * RING]
    deg_sh = rest[-1]
    c = lax.axis_index("c")
    s = lax.axis_index("s")
    w = _worker_id()
    E = eif.shape[0] // 2
    nck = E // CH // NW

    def ob(i, carry):
        ones_v[pl.ds(i * L, L)] = jnp.ones((L,), jnp.int32)
        return carry
    lax.fori_loop(0, CH // L, ob, 0)
    _zero_fill(tmp_v, PS, jnp.int32)
    pltpu.sync_copy(tmp_v, deg_sh.at[pl.ds(s * PS, PS)])
    plsc.subcore_barrier()

    lo = w * nck
    ld, sc = _Board(), _Board()

    def start_loads(g):
        b = g % RING
        e0 = (lo + g) * CH
        ld.put(g, [
            pltpu.async_copy(eif.at[pl.ds(e0, CH)], src_v[b], sem_ld[b]),
            pltpu.async_copy(eif.at[pl.ds(E + e0, CH)], dst_v[b],
                             sem_ld[b]),
        ])

    for r in range(min(RING - 1, nck)):
        start_loads(r)
    for g in range(nck):
        b = g % RING
        ld.wait(g)
        sc.put(g, [
            pltpu.async_copy(ones_v, deg_sh.at[src_v[b]], sem_sc[b],
                             add=True),
            pltpu.async_copy(ones_v, deg_sh.at[dst_v[b]], sem_sc[b],
                             add=True),
        ])
        if g + RING - 1 < nck:
            sc.wait(g - 1)
            start_loads(g + RING - 1)
    sc.drain()
    plsc.subcore_barrier()
    pltpu.sync_copy(deg_sh.at[pl.ds(s * PS, PS)], tmp_v)
    pltpu.sync_copy(tmp_v, out.at[pl.ds(c * NPAD + s * PS, PS)])


def _main_body(eif, et1, lg1, relp, ht, degp, out, kn_out, *rest):
    src_v = rest[0:RING]
    dst_v = rest[RING:2 * RING]
    et_v = rest[2 * RING:3 * RING]
    lg_v = rest[3 * RING:4 * RING]
    ks_v = rest[4 * RING:4 * RING + 2]
    msg_v = rest[4 * RING + 2:4 * RING + 4]
    rel_v, ht_v, d0_v, d1_v, kn_v = rest[4 * RING + 4:4 * RING + 9]
    p = 4 * RING + 9
    sem_ld = rest[p:p + RING]
    sem_gt = rest[p + RING:p + RING + 2]
    sem_sc = rest[p + RING + 2:p + RING + 4]
    kn_sh, acc_sh = rest[p + RING + 4:]
    c = lax.axis_index("c")
    s = lax.axis_index("s")
    w = _worker_id()
    E = et1.shape[0]
    nck = E // CH // NW

    pltpu.sync_copy(relp, rel_v)
    pltpu.sync_copy(ht, ht_v)

    off = s * PS
    pltpu.sync_copy(degp.at[pl.ds(off, PS)], d0_v)
    pltpu.sync_copy(degp.at[pl.ds(NPAD + off, PS)], d1_v)
    head = ht_v[0, :]
    tail = ht_v[1, :]

    def b1(i, carry):
        sl = pl.ds(i * L, L)
        dd = d0_v[sl] + d1_v[sl]
        nid = off + i * L + lax.iota(jnp.int32, L)
        keep = ((dd <= PRUNE_MAX_DEGREE) | (nid == head)) | (nid == tail)
        kn_v[sl] = jnp.where(keep, 1.0, 0.0).astype(jnp.float32)
        return carry

    lax.fori_loop(0, PS // L, b1, 0)
    pltpu.sync_copy(kn_v, kn_sh.at[pl.ds(off, PS)])

    @pl.when(c == 0)
    def _():
        pltpu.sync_copy(kn_v, kn_out.at[pl.ds(off, PS)])

    _zero_fill(kn_v, PS, jnp.float32)
    pltpu.sync_copy(kn_v, acc_sh.at[pl.ds(off, PS)])
    plsc.subcore_barrier()

    lo = w * nck
    ld, gt, sc = _Board(), _Board(), _Board()

    def start_loads(g):
        b = g % RING
        e0 = (lo + g) * CH
        ld.put(g, [
            pltpu.async_copy(eif.at[pl.ds(e0, CH)], src_v[b], sem_ld[b]),
            pltpu.async_copy(eif.at[pl.ds(E + e0, CH)], dst_v[b],
                             sem_ld[b]),
            pltpu.async_copy(et1.at[pl.ds(e0, CH)], et_v[b], sem_ld[b]),
            pltpu.async_copy(lg1.at[pl.ds(e0, CH)], lg_v[b], sem_ld[b]),
        ])

    def start_gathers(g):
        bi, bg = g % RING, g % 2
        gt.put(g, [
            pltpu.async_copy(kn_sh.at[src_v[bi]], ks_v[bg], sem_gt[bg]),
        ])

    for r in range(min(RING - 1, nck)):
        start_loads(r)
    if nck:
        ld.wait(0)
        start_gathers(0)
    for g in range(nck):
        bi, bg = g % RING, g % 2
        gt.wait(g)
        if g + 1 < nck:
            ld.wait(g + 1)
            start_gathers(g + 1)
        sc.wait(g - 2)

        def cb(j, carry2):
            sl = pl.ds(j * L, L)
            lg = lg_v[bi][sl]
            sig = 1.0 / (1.0 + jnp.exp(-lg))
            re = plsc.load_gather(rel_v, [et_v[bi][sl]])
            msg_v[bg][sl] = sig * ks_v[bg][sl] * re
            return carry2

        lax.fori_loop(0, CH // L, cb, 0)
        sc.put(g, [pltpu.async_copy(msg_v[bg], acc_sh.at[dst_v[bi]],
                                    sem_sc[bg], add=True)])
        if g + RING - 1 < nck:
            sc.wait(g - 1)
            start_loads(g + RING - 1)
    sc.drain()
    plsc.subcore_barrier()
    pltpu.sync_copy(acc_sh.at[pl.ds(off, PS)], kn_v)
    pltpu.sync_copy(kn_v, out.at[pl.ds(c * NPAD + off, PS)])


def _combine_body(p_ref, k_ref, o_ref):
    o_ref[...] = (p_ref[0] + p_ref[1]) * k_ref[...]


def kernel(edge_index, edge_type, edge_mask_logits, rel_emb, head_idx, tail_idx):
    E = edge_type.shape[0]
    R = rel_emb.shape[0]
    assert E % (CH * NW) == 0
    eif = edge_index.reshape(2 * E)

    rpad = ((R + L - 1) // L) * L
    relp = jnp.zeros((rpad,), jnp.float32).at[:R].set(rel_emb)
    ht = jnp.stack([jnp.full((L,), head_idx, jnp.int32),
                    jnp.full((L,), tail_idx, jnp.int32)])

    mesh = plsc.VectorSubcoreMesh(core_axis_name="c", subcore_axis_name="s")

    deg_partial = pl.kernel(
        _deg_body,
        out_type=jax.ShapeDtypeStruct((NC * NPAD,), jnp.int32),
        mesh=mesh,
        scratch_types=(
            [pltpu.VMEM((CH,), jnp.int32)] * RING
            + [pltpu.VMEM((CH,), jnp.int32)] * RING
            + [
                pltpu.VMEM((CH,), jnp.int32),
                pltpu.VMEM((PS,), jnp.int32),
            ]
            + [pltpu.SemaphoreType.DMA] * RING
            + [pltpu.SemaphoreType.DMA] * RING
            + [pltpu.VMEM_SHARED((NPAD,), jnp.int32)]
        ),
    )(eif)

    out_partial, kn_arr = pl.kernel(
        _main_body,
        out_type=(jax.ShapeDtypeStruct((NC * NPAD,), jnp.float32),
                  jax.ShapeDtypeStruct((NPAD,), jnp.float32)),
        mesh=mesh,
        scratch_types=(
            [pltpu.VMEM((CH,), jnp.int32)] * RING
            + [pltpu.VMEM((CH,), jnp.int32)] * RING
            + [pltpu.VMEM((CH,), jnp.int32)] * RING
            + [pltpu.VMEM((CH,), jnp.float32)] * RING
            + [pltpu.VMEM((CH,), jnp.float32)] * 2
            + [pltpu.VMEM((CH,), jnp.float32)] * 2
            + [
                pltpu.VMEM((rpad,), jnp.float32),
                pltpu.VMEM((2, L), jnp.int32),
                pltpu.VMEM((PS,), jnp.int32),
                pltpu.VMEM((PS,), jnp.int32),
                pltpu.VMEM((PS,), jnp.float32),
            ]
            + [pltpu.SemaphoreType.DMA] * RING
            + [pltpu.SemaphoreType.DMA] * 2
            + [pltpu.SemaphoreType.DMA] * 2
            + [pltpu.VMEM_SHARED((NPAD,), jnp.float32),
               pltpu.VMEM_SHARED((NPAD,), jnp.float32)]
        ),
        compiler_params=pltpu.CompilerParams(needs_layout_passes=False),
    )(eif, edge_type, edge_mask_logits, relp, ht, deg_partial)

    out = pl.pallas_call(
        _combine_body,
        out_shape=jax.ShapeDtypeStruct((NPAD // ROW, ROW), jnp.float32),
    )(out_partial.reshape(NC, NPAD // ROW, ROW),
      kn_arr.reshape(NPAD // ROW, ROW))
    return out.reshape(-1)[:N]
